# baseline (speedup 1.0000x reference)
import functools

import jax
import jax.numpy as jnp
from jax import lax
from jax.experimental import pallas as pl
from jax.experimental.pallas import tpu as pltpu
from jax.experimental.pallas import tpu_sc as plsc

_E = 8
_C = 2
_FT = 512
_L = 16


def _gate_body(x_ref, wg_ref, st_ref):
    logits = lax.dot_general(wg_ref[...], x_ref[...],
                             (((0,), (1,)), ((), ())),
                             preferred_element_type=jnp.float32)
    st_ref[...] = jax.nn.sigmoid(logits)


def _gate_tc(xf, Wg):
    T, D = xf.shape
    E = Wg.shape[1]
    return pl.pallas_call(
        _gate_body,
        out_shape=jax.ShapeDtypeStruct((E, T), jnp.float32),
    )(xf, Wg)


def _topk_sc_body(st_hbm, i_hbm, g_hbm, s_v, idx_v, gv_v, *, E, T, K):
    wid = lax.axis_index("s") * 2 + lax.axis_index("c")
    NV = T // _L

    @pl.when(wid < E)
    def _():
        pltpu.sync_copy(st_hbm.at[wid], s_v)

        def count_ge(th):
            def body(j, acc):
                v = lax.bitcast_convert_type(s_v[pl.ds(j * _L, _L)], jnp.int32)
                return acc + jnp.where(v >= th, 1, 0).astype(jnp.int32)
            acc = lax.fori_loop(0, NV, body, jnp.zeros((_L,), jnp.int32),
                                unroll=4)
            return plsc.cumsum(acc)[_L - 1]

        def bis(_, lohi):
            lo, hi = lohi
            mid = (lo + hi) // 2
            ok = count_ge(mid) >= K
            return jnp.where(ok, mid, lo), jnp.where(ok, hi, mid)

        vbits, _ = lax.fori_loop(0, 31, bis,
                                 (jnp.int32(0), jnp.int32(0x3F800001)))
        need_eq = K - count_ge(vbits + 1)

        def comp(j, carry):
            off, neq = carry
            v = s_v[pl.ds(j * _L, _L)]
            vb = lax.bitcast_convert_type(v, jnp.int32)
            m_gt = vb > vbits
            m_eq = vb == vbits
            incl = plsc.cumsum(m_eq.astype(jnp.int32))
            m_take = jnp.logical_and(m_eq, incl <= neq)
            m = jnp.logical_or(m_gt, m_take)
            ids = lax.broadcasted_iota(jnp.int32, (_L,), 0) + j * _L
            plsc.store_compressed(idx_v.at[pl.ds(off, _L)], ids, mask=m)
            plsc.store_compressed(gv_v.at[pl.ds(off, _L)], v, mask=m)
            nsel = plsc.all_reduce_population_count(m)[0]
            ntak = plsc.all_reduce_population_count(m_take)[0]
            return off + nsel, neq - ntak

        lax.fori_loop(0, NV, comp, (jnp.int32(0), need_eq), unroll=2)
        pltpu.sync_copy(idx_v.at[pl.ds(0, K)], i_hbm.at[wid])
        pltpu.sync_copy(gv_v.at[pl.ds(0, K)], g_hbm.at[wid])


def _topk_sc(St, K):
    E, T = St.shape
    mesh = plsc.VectorSubcoreMesh(core_axis_name="c", subcore_axis_name="s")
    body = functools.partial(_topk_sc_body, E=E, T=T, K=K)
    f = pl.kernel(
        body,
        out_type=[jax.ShapeDtypeStruct((E, K), jnp.int32),
                  jax.ShapeDtypeStruct((E, K), jnp.float32)],
        mesh=mesh,
        scratch_types=[pltpu.VMEM((T,), jnp.float32),
                       pltpu.VMEM((K + _L,), jnp.int32),
                       pltpu.VMEM((K + _L,), jnp.float32)],
        compiler_params=pltpu.CompilerParams(needs_layout_passes=False),
    )
    return f(St)


def _moe_body(x_ref, w1_ref, b1_ref, w2_ref, b2_ref, g_ref, i_ref, out_ref,
              xe_ref, yacc_ref, *, T, K, F):
    e = pl.program_id(0)
    f = pl.program_id(1)

    @pl.when(f == 0)
    def _gather():
        idx = i_ref[0, 0, :]
        tok = jax.lax.broadcasted_iota(jnp.int32, (K, T), 1)
        p = (idx[:, None] == tok).astype(jnp.bfloat16)
        xe = jnp.dot(p, x_ref[...].astype(jnp.bfloat16),
                     preferred_element_type=jnp.float32)
        xe_ref[...] = xe.astype(jnp.bfloat16)

    h = jnp.dot(xe_ref[...], w1_ref[0].astype(jnp.bfloat16),
                preferred_element_type=jnp.float32)
    h = jax.nn.gelu(h + b1_ref[0, 0][None, :], approximate=True)
    y = jnp.dot(h.astype(jnp.bfloat16), w2_ref[0].astype(jnp.bfloat16),
                preferred_element_type=jnp.float32)

    @pl.when(f == 0)
    def _init_yacc():
        yacc_ref[...] = jnp.zeros_like(yacc_ref)

    yacc_ref[...] += y

    @pl.when(jnp.logical_and(e == 0, f == 0))
    def _init_out():
        out_ref[...] = jnp.zeros_like(out_ref)

    @pl.when(f == F - 1)
    def _scatter():
        idx = i_ref[0, 0, :]
        yk = g_ref[0, 0, :][:, None] * (yacc_ref[...] + b2_ref[0, 0][None, :])
        tok = jax.lax.broadcasted_iota(jnp.int32, (T, K), 0)
        pt = (tok == idx[None, :]).astype(jnp.bfloat16)
        out_ref[...] += jnp.dot(pt, yk.astype(jnp.bfloat16),
                                preferred_element_type=jnp.float32)


def _moe_tc(xf, W1, b1, W2, b2, G, I, *, interpret=False):
    T, D = xf.shape
    E, _, DFF = W1.shape
    K = G.shape[-1]
    F = DFF // _FT
    grid = (E, F)
    body = functools.partial(_moe_body, T=T, K=K, F=F)
    return pl.pallas_call(
        body,
        grid=grid,
        in_specs=[
            pl.BlockSpec((T, D), lambda e, f: (0, 0)),
            pl.BlockSpec((1, D, _FT), lambda e, f: (e, 0, f)),
            pl.BlockSpec((1, 1, _FT), lambda e, f: (e * F + f, 0, 0)),
            pl.BlockSpec((1, _FT, D), lambda e, f: (e, f, 0)),
            pl.BlockSpec((1, 1, D), lambda e, f: (e, 0, 0)),
            pl.BlockSpec((1, 1, K), lambda e, f: (e, 0, 0)),
            pl.BlockSpec((1, 1, K), lambda e, f: (e, 0, 0)),
        ],
        out_specs=pl.BlockSpec((T, D), lambda e, f: (0, 0)),
        out_shape=jax.ShapeDtypeStruct((T, D), jnp.float32),
        scratch_shapes=[
            pltpu.VMEM((K, D), jnp.bfloat16),
            pltpu.VMEM((K, D), jnp.float32),
        ],
        compiler_params=pltpu.CompilerParams(
            dimension_semantics=("arbitrary", "arbitrary"),
        ),
        interpret=interpret,
    )(xf, W1, b1.reshape(E * F, 1, _FT), W2, b2.reshape(E, 1, D), G, I)


def kernel(x, Wg, W1, b1, W2, b2):
    b, l, d = x.shape
    xf = x.reshape(b * l, d)
    T = b * l
    E = W1.shape[0]
    k = min(max(int(T * _C / E), 1), T)
    St = _gate_tc(xf, Wg)
    I, G = _topk_sc(St, k)
    out = _moe_tc(xf, W1, b1, W2, b2,
                  G.reshape(E, 1, k), I.reshape(E, 1, k))
    return out.reshape(b, l, d)
